# Initial kernel scaffold; baseline (speedup 1.0000x reference)
import jax
import jax.numpy as jnp
from jax.experimental import pallas as pl


def kernel(x, adj, W, b):
    raise NotImplementedError("write your pallas kernel here")


# speedup vs baseline: 1.9014x; 1.9014x over previous
import jax
import jax.numpy as jnp
from jax.experimental import pallas as pl

_N = 10000
_F = 128
_R = 400


def _sage_body(adj_ref, x_ref, xs_ref, ws_ref, wa_ref, b_ref, out_ref):
    a = adj_ref[...]
    deg = jnp.maximum(jnp.sum(a, axis=1, keepdims=True), 1.0)
    acc = jnp.dot(a, x_ref[...], preferred_element_type=jnp.float32)
    agg = acc / deg
    h = (
        jnp.dot(xs_ref[...], ws_ref[...], preferred_element_type=jnp.float32)
        + jnp.dot(agg, wa_ref[...], preferred_element_type=jnp.float32)
        + b_ref[...]
    )
    out_ref[...] = jnp.maximum(h, 0.0)


def kernel(x, adj, W, b):
    wt = W.T
    ws, wa = wt[:_F], wt[_F:]
    b2 = b.reshape(1, _F)
    return pl.pallas_call(
        _sage_body,
        grid=(_N // _R,),
        in_specs=[
            pl.BlockSpec((_R, _N), lambda i: (i, 0)),
            pl.BlockSpec((_N, _F), lambda i: (0, 0)),
            pl.BlockSpec((_R, _F), lambda i: (i, 0)),
            pl.BlockSpec((_F, _F), lambda i: (0, 0)),
            pl.BlockSpec((_F, _F), lambda i: (0, 0)),
            pl.BlockSpec((1, _F), lambda i: (0, 0)),
        ],
        out_specs=pl.BlockSpec((_R, _F), lambda i: (i, 0)),
        out_shape=jax.ShapeDtypeStruct((_N, _F), jnp.float32),
    )(adj, x, x, ws, wa, b2)
